# baseline (speedup 1.0000x reference)
import functools

import jax
import jax.numpy as jnp
from jax import lax
from jax.experimental import pallas as pl
from jax.experimental.pallas import tpu as pltpu
from jax.experimental.pallas import tpu_sc as plsc

VOCAB = 100000
BATCH_DIM = 16384
SEQ = 7
TOTAL = BATCH_DIM * SEQ
NC, NS = 2, 16
NW = NC * NS
PER_W = TOTAL // NW
CHUNK = 128
NCHUNK = PER_W // CHUNK

_mesh = plsc.VectorSubcoreMesh(core_axis_name="c", subcore_axis_name="s")


@functools.partial(
    pl.kernel,
    mesh=_mesh,
    out_type=jax.ShapeDtypeStruct((TOTAL,), jnp.float32),
    scratch_types=[
        pltpu.VMEM((PER_W,), jnp.int32),
        pltpu.VMEM((PER_W,), jnp.float32),
        pltpu.VMEM_SHARED((VOCAB,), jnp.float32),
        pltpu.SemaphoreType.DMA,
        pltpu.SemaphoreType.DMA,
    ],
)
def _gather(idx_hbm, table_hbm, out_hbm, idx_v, rows_v, table_sh, sem, sem2):
    sid = lax.axis_index("s")
    cid = lax.axis_index("c")
    base = (sid * NC + cid) * PER_W

    @pl.when(sid == 0)
    def _():
        pltpu.sync_copy(table_hbm, table_sh)

    pltpu.sync_copy(idx_hbm.at[pl.ds(base, PER_W)], idx_v)
    plsc.subcore_barrier()
    hbm_n = 1152
    sp_n = PER_W - hbm_n
    c_hbm = pltpu.async_copy(
        table_hbm.at[idx_v.at[pl.ds(0, hbm_n)]],
        rows_v.at[pl.ds(0, hbm_n)],
        sem,
    )
    c_sp = pltpu.async_copy(
        table_sh.at[idx_v.at[pl.ds(hbm_n, sp_n)]],
        rows_v.at[pl.ds(hbm_n, sp_n)],
        sem2,
    )
    c_hbm.wait()
    c_sp.wait()
    pltpu.sync_copy(rows_v, out_hbm.at[pl.ds(base, PER_W)])


def kernel(faked_id, lookup_values):
    idx_flat = faked_id.reshape(TOTAL)
    out = _gather(idx_flat, lookup_values)
    return out.reshape(BATCH_DIM, SEQ)


# speedup vs baseline: 1.0334x; 1.0334x over previous
import functools

import jax
import jax.numpy as jnp
from jax import lax
from jax.experimental import pallas as pl
from jax.experimental.pallas import tpu as pltpu
from jax.experimental.pallas import tpu_sc as plsc

VOCAB = 100000
BATCH_DIM = 16384
SEQ = 7
TOTAL = BATCH_DIM * SEQ
NC, NS = 2, 16
NW = NC * NS
PER_W = TOTAL // NW

_mesh = plsc.VectorSubcoreMesh(core_axis_name="c", subcore_axis_name="s")


@functools.partial(
    pl.kernel,
    mesh=_mesh,
    out_type=jax.ShapeDtypeStruct((TOTAL,), jnp.float32),
    scratch_types=[
        pltpu.VMEM((PER_W,), jnp.int32),
        pltpu.VMEM((PER_W,), jnp.float32),
        pltpu.VMEM_SHARED((VOCAB,), jnp.float32),
        pltpu.SemaphoreType.DMA,
        pltpu.SemaphoreType.DMA,
    ],
)
def _gather(idx_hbm, table_hbm, out_hbm, idx_v, rows_v, table_sh, sem, sem2):
    sid = lax.axis_index("s")
    cid = lax.axis_index("c")
    base = (sid * NC + cid) * PER_W

    @pl.when(sid == 0)
    def _():
        stage = pltpu.async_copy(table_hbm, table_sh, sem2)
        pltpu.sync_copy(idx_hbm.at[pl.ds(base, PER_W)], idx_v)
        stage.wait()

    @pl.when(sid != 0)
    def _():
        pltpu.sync_copy(idx_hbm.at[pl.ds(base, PER_W)], idx_v)

    plsc.subcore_barrier()
    pltpu.async_copy(table_sh.at[idx_v], rows_v, sem).wait()
    pltpu.sync_copy(rows_v, out_hbm.at[pl.ds(base, PER_W)])


def kernel(faked_id, lookup_values):
    idx_flat = faked_id.reshape(TOTAL)
    out = _gather(idx_flat, lookup_values)
    return out.reshape(BATCH_DIM, SEQ)
